# baseline (speedup 1.0000x reference)
import numpy as np
import jax
import jax.numpy as jnp
from jax.experimental import pallas as pl
from jax.experimental.pallas import tpu as pltpu

SR = 44100
N_FFT = 2048
HOP = 1024
N_BINS = N_FFT // 2 + 1
FRAME_BLOCK = 128


def _a_weight_curve_np():
    freqs = np.fft.rfftfreq(N_FFT, 1.0 / SR).astype(np.float32)
    f2 = freqs * freqs
    c1 = np.float32(20.6 ** 2)
    c2 = np.float32(107.7 ** 2)
    c3 = np.float32(737.9 ** 2)
    c4 = np.float32(12194.0 ** 2)
    num = c4 * f2 * f2
    den = (f2 + c1) * np.sqrt((f2 + c2) * (f2 + c3)) * (f2 + c4)
    return num / np.maximum(den, np.float32(1e-12))


def _tables_np():
    n = np.arange(HOP, dtype=np.float64)[:, None]
    k = np.arange(N_BINS, dtype=np.float64)[None, :]
    th = 2.0 * np.pi / N_FFT
    ra = _a_weight_curve_np().astype(np.float64)[None, :]
    cos_t = np.cos(th * n * k) * ra
    sin_t = np.sin(th * n * k) * ra
    cos_t[0, :] = np.cos(np.pi * k[0, :]) * ra[0, :]
    sin_t[0, :] = 0.0
    return cos_t.astype(jnp.bfloat16), sin_t.astype(jnp.bfloat16)


_COS_T, _SIN_T = _tables_np()
_REV128 = np.eye(128, dtype=np.float32)[:, ::-1].astype(jnp.bfloat16)


def _stft_block(x_ref, halo_ref, wlo_ref, ws_ref, c_ref, s_ref,
                q_ref, o_ref):
    n_ch = x_ref.shape[0]
    rows = n_ch * FRAME_BLOCK
    hi = x_ref[...].reshape(rows, HOP)
    prev_rows = halo_ref[...]
    prev_exp = jnp.broadcast_to(
        prev_rows[:, None, :], (n_ch, FRAME_BLOCK, HOP)).reshape(rows, HOP)
    rolled = pltpu.roll(hi, 1, axis=0)
    row_id = jax.lax.broadcasted_iota(jnp.int32, (rows, HOP), 0)
    lo = jnp.where(row_id % FRAME_BLOCK == 0, prev_exp, rolled)
    hb = hi.astype(jnp.bfloat16)
    q = q_ref[...]
    parts = [
        jax.lax.dot_general(
            hb[:, 128 * a:128 * (a + 1)], q, (((1,), (0,)), ((), ())),
            preferred_element_type=jnp.float32)
        for a in range(HOP // 128)
    ]
    flip = jnp.concatenate(parts[::-1], axis=1)
    s = pltpu.roll(flip, 1, axis=1)
    wlo = wlo_ref[0, :][None, :]
    ws = ws_ref[0, :][None, :]
    lo_w = lo * wlo
    s_w = s * ws
    e = (lo_w + s_w).astype(jnp.bfloat16)
    o = (lo_w - s_w).astype(jnp.bfloat16)
    re = jax.lax.dot_general(
        e, c_ref[...], (((1,), (0,)), ((), ())),
        preferred_element_type=jnp.float32)
    im = jax.lax.dot_general(
        o, s_ref[...], (((1,), (0,)), ((), ())),
        preferred_element_type=jnp.float32)
    o_ref[...] = (re * re + im * im).reshape(n_ch, FRAME_BLOCK, N_BINS)


def kernel(x, cache, window):
    n_ch, n_samples = x.shape
    n_frames = (n_samples + cache.shape[1] - N_FFT) // HOP + 1
    n_steps = n_frames // FRAME_BLOCK
    halo = jnp.concatenate(
        [cache] + [x[:, (j * FRAME_BLOCK - 1) * HOP:j * FRAME_BLOCK * HOP]
                   for j in range(1, n_steps)], axis=1)
    wlo = window[:HOP].reshape(1, HOP)
    ws = jnp.concatenate([window[HOP:HOP + 1], window[1:HOP]]).reshape(1, HOP)
    cos_t = jnp.asarray(_COS_T)
    sin_t = jnp.asarray(_SIN_T)

    grid = (n_frames // FRAME_BLOCK,)
    out = pl.pallas_call(
        _stft_block,
        grid=grid,
        in_specs=[
            pl.BlockSpec((n_ch, FRAME_BLOCK * HOP), lambda j: (0, j)),
            pl.BlockSpec((n_ch, HOP), lambda j: (0, j)),
            pl.BlockSpec((1, HOP), lambda j: (0, 0)),
            pl.BlockSpec((1, HOP), lambda j: (0, 0)),
            pl.BlockSpec((HOP, N_BINS), lambda j: (0, 0)),
            pl.BlockSpec((HOP, N_BINS), lambda j: (0, 0)),
            pl.BlockSpec((128, 128), lambda j: (0, 0)),
        ],
        out_specs=pl.BlockSpec(
            (n_ch, FRAME_BLOCK, N_BINS), lambda j: (0, j, 0)),
        out_shape=jax.ShapeDtypeStruct((n_ch, n_frames, N_BINS), jnp.float32),
    )(x, halo, wlo, ws, cos_t, sin_t, jnp.asarray(_REV128))
    return out


# speedup vs baseline: 1.3397x; 1.3397x over previous
import numpy as np
import jax
import jax.numpy as jnp
from jax.experimental import pallas as pl
from jax.experimental.pallas import tpu as pltpu

SR = 44100
N_FFT = 2048
HOP = 1024
N_BINS = N_FFT // 2 + 1
FRAME_BLOCK = 128


def _a_weight_curve_np():
    freqs = np.fft.rfftfreq(N_FFT, 1.0 / SR).astype(np.float32)
    f2 = freqs * freqs
    c1 = np.float32(20.6 ** 2)
    c2 = np.float32(107.7 ** 2)
    c3 = np.float32(737.9 ** 2)
    c4 = np.float32(12194.0 ** 2)
    num = c4 * f2 * f2
    den = (f2 + c1) * np.sqrt((f2 + c2) * (f2 + c3)) * (f2 + c4)
    return num / np.maximum(den, np.float32(1e-12))


def _tables_np():
    n = np.arange(HOP, dtype=np.float64)[:, None]
    k = np.arange(N_BINS, dtype=np.float64)[None, :]
    th = 2.0 * np.pi / N_FFT
    ra = _a_weight_curve_np().astype(np.float64)[None, :]
    cos_t = np.cos(th * n * k) * ra
    sin_t = np.sin(th * n * k) * ra
    cos_t[0, :] = np.cos(np.pi * k[0, :]) * ra[0, :]
    sin_t[0, :] = 0.0
    return cos_t.astype(jnp.bfloat16), sin_t.astype(jnp.bfloat16)


_COS_T, _SIN_T = _tables_np()
_REV128 = np.eye(128, dtype=np.float32)[:, ::-1].astype(jnp.bfloat16)


def _stft_block(x_ref, prev_ref, cache_ref, wlo_ref, ws_ref, c_ref, s_ref,
                q_ref, o_ref):
    j = pl.program_id(0)
    n_ch = x_ref.shape[0]
    rows = n_ch * FRAME_BLOCK
    hi = x_ref[...].reshape(rows, HOP)
    prev_rows = jnp.where(j == 0, cache_ref[...], prev_ref[...])
    prev_exp = jnp.broadcast_to(
        prev_rows[:, None, :], (n_ch, FRAME_BLOCK, HOP)).reshape(rows, HOP)
    rolled = pltpu.roll(hi, 1, axis=0)
    row_id = jax.lax.broadcasted_iota(jnp.int32, (rows, HOP), 0)
    lo = jnp.where(row_id % FRAME_BLOCK == 0, prev_exp, rolled)
    hb = hi.astype(jnp.bfloat16)
    q = q_ref[...]
    parts = [
        jax.lax.dot_general(
            hb[:, 128 * a:128 * (a + 1)], q, (((1,), (0,)), ((), ())),
            preferred_element_type=jnp.float32)
        for a in range(HOP // 128)
    ]
    flip = jnp.concatenate(parts[::-1], axis=1)
    s = pltpu.roll(flip, 1, axis=1)
    wlo = wlo_ref[0, :][None, :]
    ws = ws_ref[0, :][None, :]
    lo_w = lo * wlo
    s_w = s * ws
    e = (lo_w + s_w).astype(jnp.bfloat16)
    o = (lo_w - s_w).astype(jnp.bfloat16)
    re = jax.lax.dot_general(
        e, c_ref[...], (((1,), (0,)), ((), ())),
        preferred_element_type=jnp.float32)
    im = jax.lax.dot_general(
        o, s_ref[...], (((1,), (0,)), ((), ())),
        preferred_element_type=jnp.float32)
    p = re * re + im * im
    for c in range(n_ch):
        o_ref[:, c, :] = p[c * FRAME_BLOCK:(c + 1) * FRAME_BLOCK, :].T


def kernel(x, cache, window):
    n_ch, n_samples = x.shape
    n_frames = (n_samples + cache.shape[1] - N_FFT) // HOP + 1
    wlo = window[:HOP].reshape(1, HOP)
    ws = jnp.concatenate([window[HOP:HOP + 1], window[1:HOP]]).reshape(1, HOP)
    cos_t = jnp.asarray(_COS_T)
    sin_t = jnp.asarray(_SIN_T)

    grid = (n_frames // FRAME_BLOCK,)
    out = pl.pallas_call(
        _stft_block,
        grid=grid,
        in_specs=[
            pl.BlockSpec((n_ch, FRAME_BLOCK * HOP), lambda j: (0, j)),
            pl.BlockSpec(
                (n_ch, HOP),
                lambda j: (0, jnp.maximum(j * FRAME_BLOCK - 1, 0))),
            pl.BlockSpec((n_ch, HOP), lambda j: (0, 0)),
            pl.BlockSpec((1, HOP), lambda j: (0, 0)),
            pl.BlockSpec((1, HOP), lambda j: (0, 0)),
            pl.BlockSpec((HOP, N_BINS), lambda j: (0, 0)),
            pl.BlockSpec((HOP, N_BINS), lambda j: (0, 0)),
            pl.BlockSpec((128, 128), lambda j: (0, 0)),
        ],
        out_specs=pl.BlockSpec(
            (N_BINS, n_ch, FRAME_BLOCK), lambda j: (0, 0, j)),
        out_shape=jax.ShapeDtypeStruct((N_BINS, n_ch, n_frames), jnp.float32),
    )(x, x, cache, wlo, ws, cos_t, sin_t, jnp.asarray(_REV128))
    return jnp.transpose(out, (1, 2, 0))
